# baseline (speedup 1.0000x reference)
import math

import numpy as np

import jax
import jax.numpy as jnp
from jax.experimental import pallas as pl
from jax.experimental.pallas import tpu as pltpu

_VMEM_LIMIT = 64 * 1024 * 1024
_DB = 8


def _interp_mat(n_in, n_out):
    m = np.zeros((n_out, n_in), np.float32)
    for i in range(n_out):
        src = 0.0 if n_out == 1 else i * (n_in - 1) / (n_out - 1)
        i0 = min(int(math.floor(src)), n_in - 1)
        i1 = min(i0 + 1, n_in - 1)
        f = src - i0
        m[i, i0] += 1.0 - f
        m[i, i1] += f
    return m


def _store_variants(vs_ref, j, base, HW, W, C):
    wi = jax.lax.broadcasted_iota(jnp.int32, (HW, 1), 0) % W
    zrow = jnp.zeros((1, C), base.dtype)
    lo = j * HW
    vs_ref[0, lo:lo + HW, :] = jnp.where(
        wi >= 1, jnp.concatenate([zrow, base[:HW - 1]], axis=0),
        jnp.zeros_like(base))
    vs_ref[1, lo:lo + HW, :] = base
    vs_ref[2, lo:lo + HW, :] = jnp.where(
        wi <= W - 2, jnp.concatenate([base[1:], zrow], axis=0),
        jnp.zeros_like(base))


def _zero_halo_slices(vs_ref, c, nc, HW, C, DB):
    @pl.when(c == 0)
    def _():
        vs_ref[:, 0:HW, :] = jnp.zeros((3, HW, C), vs_ref.dtype)

    @pl.when(c == nc - 1)
    def _():
        lo = (DB + 1) * HW
        vs_ref[:, lo:lo + HW, :] = jnp.zeros((3, HW, C), vs_ref.dtype)


def _chunk_conv(acc_ref, vs_ref, w_ref, HW, W, DB):
    M = DB * HW
    for kd in range(3):
        for kh in range(3):
            s = (kh - 1) * W
            y = None
            for kw in range(3):
                op = vs_ref[kw, kd * HW:kd * HW + M, :]
                p = jnp.dot(op, w_ref[kd * 9 + kh * 3 + kw],
                            preferred_element_type=jnp.float32)
                y = p if y is None else y + p
            for k in range(DB):
                a = k * HW
                if s == 0:
                    acc_ref[a:a + HW, :] += y[a:a + HW, :]
                elif s > 0:
                    acc_ref[a:a + HW - s, :] += y[a + s:a + HW, :]
                else:
                    acc_ref[a - s:a + HW, :] += y[a:a + HW + s, :]


def _make_up_conv_body(Din, Dout, Hin, Win, H, W, Cu, Cs, Cm, DB):
    HW = H * W
    Cin = Cu + Cs

    def body(*refs):
        x1_ref = refs[0]
        x2_refs = refs[1:DB + 3]
        mhw_ref, w_ref, b_ref, o_ref, vs_ref, acc_ref = refs[DB + 3:]
        c = pl.program_id(1)
        nc = pl.num_programs(1)
        acc_ref[...] = jnp.zeros(acc_ref.shape, jnp.float32)

        for j in range(DB + 2):
            od = jnp.clip(c * DB + (j - 1), 0, Dout - 1)
            t = od * (Din - 1)
            i0 = t // (Dout - 1)
            i1 = jnp.minimum(i0 + 1, Din - 1)
            fd = (t % (Dout - 1)).astype(jnp.float32) * (1.0 / (Dout - 1))
            a0 = x1_ref[0, pl.ds(i0, 1)].reshape(Hin * Win, Cu)
            a1 = x1_ref[0, pl.ds(i1, 1)].reshape(Hin * Win, Cu)
            xz = a0 + fd.astype(jnp.bfloat16) * (a1 - a0)
            u = jnp.dot(mhw_ref[...], xz,
                        preferred_element_type=jnp.float32)
            base = jnp.concatenate(
                [u.astype(jnp.bfloat16), x2_refs[j][...].reshape(HW, Cs)],
                axis=1)
            _store_variants(vs_ref, j, base, HW, W, Cin)

        _zero_halo_slices(vs_ref, c, nc, HW, Cin, DB)
        _chunk_conv(acc_ref, vs_ref, w_ref, HW, W, DB)

        y = jnp.maximum(acc_ref[...] + b_ref[...], 0.0)
        o_ref[...] = y.reshape(1, DB, H, W, Cm).astype(o_ref.dtype)

    return body


def _up_conv1(x1b, x2b, w1t, b1, mhw, DB):
    N, Din, Hin, Win, Cu = x1b.shape
    _, Dout, H, W, Cs = x2b.shape
    Cm = w1t.shape[-1]
    nc = Dout // DB

    def sm(j):
        return lambda n, c: (n, jnp.clip(c * DB + (j - 1), 0, Dout - 1),
                             0, 0, 0)

    in_specs = [pl.BlockSpec((1, Din, Hin, Win, Cu),
                             lambda n, c: (n, 0, 0, 0, 0))]
    args = [x1b]
    for j in range(DB + 2):
        in_specs.append(pl.BlockSpec((1, 1, H, W, Cs), sm(j)))
        args.append(x2b)
    in_specs += [
        pl.BlockSpec((H * W, Hin * Win), lambda n, c: (0, 0)),
        pl.BlockSpec(w1t.shape, lambda n, c: (0, 0, 0)),
        pl.BlockSpec((1, Cm), lambda n, c: (0, 0)),
    ]
    args += [mhw, w1t, b1.reshape(1, Cm)]

    return pl.pallas_call(
        _make_up_conv_body(Din, Dout, Hin, Win, H, W, Cu, Cs, Cm, DB),
        out_shape=jax.ShapeDtypeStruct((N, Dout, H, W, Cm), jnp.bfloat16),
        grid=(N, nc),
        in_specs=in_specs,
        out_specs=pl.BlockSpec((1, DB, H, W, Cm),
                               lambda n, c: (n, c, 0, 0, 0)),
        scratch_shapes=[
            pltpu.VMEM((3, (DB + 2) * H * W, Cu + Cs), jnp.bfloat16),
            pltpu.VMEM((DB * H * W, Cm), jnp.float32),
        ],
        compiler_params=pltpu.CompilerParams(
            dimension_semantics=("parallel", "parallel"),
            vmem_limit_bytes=_VMEM_LIMIT),
    )(*args)


def _make_conv_body(H, W, Cin, Cout, DB):
    HW = H * W

    def body(*refs):
        x_refs = refs[:DB + 2]
        w_ref, b_ref, o_ref, vs_ref, acc_ref = refs[DB + 2:]
        c = pl.program_id(1)
        nc = pl.num_programs(1)
        acc_ref[...] = jnp.zeros(acc_ref.shape, jnp.float32)

        for j in range(DB + 2):
            _store_variants(vs_ref, j, x_refs[j][...].reshape(HW, Cin),
                            HW, W, Cin)

        _zero_halo_slices(vs_ref, c, nc, HW, Cin, DB)
        _chunk_conv(acc_ref, vs_ref, w_ref, HW, W, DB)

        y = jnp.maximum(acc_ref[...] + b_ref[...], 0.0)
        o_ref[...] = y.reshape(1, DB, H, W, Cout).astype(o_ref.dtype)

    return body


def _conv2(h, w2t, b2, out_dtype, DB):
    N, D, H, W, Cin = h.shape
    Cout = w2t.shape[-1]
    nc = D // DB

    def sm(j):
        return lambda n, c: (n, jnp.clip(c * DB + (j - 1), 0, D - 1), 0, 0, 0)

    in_specs = [pl.BlockSpec((1, 1, H, W, Cin), sm(j)) for j in range(DB + 2)]
    args = [h] * (DB + 2)
    in_specs += [
        pl.BlockSpec(w2t.shape, lambda n, c: (0, 0, 0)),
        pl.BlockSpec((1, Cout), lambda n, c: (0, 0)),
    ]
    args += [w2t, b2.reshape(1, Cout)]

    return pl.pallas_call(
        _make_conv_body(H, W, Cin, Cout, DB),
        out_shape=jax.ShapeDtypeStruct((N, D, H, W, Cout), out_dtype),
        grid=(N, nc),
        in_specs=in_specs,
        out_specs=pl.BlockSpec((1, DB, H, W, Cout),
                               lambda n, c: (n, c, 0, 0, 0)),
        scratch_shapes=[
            pltpu.VMEM((3, (DB + 2) * H * W, Cin), jnp.bfloat16),
            pltpu.VMEM((DB * H * W, Cout), jnp.float32),
        ],
        compiler_params=pltpu.CompilerParams(
            dimension_semantics=("parallel", "parallel"),
            vmem_limit_bytes=_VMEM_LIMIT),
    )(*args)


def kernel(x1, x2, w1, b1, w2, b2):
    N, Cu, Din, Hin, Win = x1.shape
    Cs, Dout, H, W = x2.shape[1], x2.shape[2], x2.shape[3], x2.shape[4]
    Cm = w1.shape[0]
    db = _DB if Dout % _DB == 0 else 1

    x1b = jnp.transpose(x1, (0, 2, 3, 4, 1)).astype(jnp.bfloat16)
    x2b = jnp.transpose(x2, (0, 2, 3, 4, 1)).astype(jnp.bfloat16)
    w1t = jnp.transpose(w1, (2, 3, 4, 1, 0)).reshape(
        27, Cu + Cs, Cm).astype(jnp.bfloat16)
    w2t = jnp.transpose(w2, (2, 3, 4, 1, 0)).reshape(
        27, Cm, w2.shape[0]).astype(jnp.bfloat16)
    mhw = jnp.asarray(np.kron(_interp_mat(Hin, H), _interp_mat(Win, W)),
                      jnp.bfloat16)

    h = _up_conv1(x1b, x2b, w1t, b1, mhw, db)
    y = _conv2(h, w2t, b2, x1.dtype, db)
    return jnp.transpose(y, (0, 4, 1, 2, 3))
